# Initial kernel scaffold; baseline (speedup 1.0000x reference)
import functools

import jax
import jax.numpy as jnp
from jax import lax
from jax.experimental import pallas as pl
from jax.experimental.pallas import tpu as pltpu
from jax.experimental.pallas import tpu_sc as plsc

_NUM_EMBEDDINGS = 1000000
_DIM = 64
_B, _S = 16384, 50
_TOTAL = _B * _S
_NC, _NS = 2, 16
_NW = _NC * _NS
_BPW = _TOTAL // _NW
_G = 128
_NG = _BPW // _G
_LANES = 16

_mesh = plsc.VectorSubcoreMesh(core_axis_name="c", subcore_axis_name="s")


@functools.partial(
    pl.kernel,
    mesh=_mesh,
    out_type=jax.ShapeDtypeStruct((_TOTAL, _DIM), jnp.float32),
    scratch_types=[
        pltpu.VMEM((_NG, _G), jnp.int32),
        pltpu.VMEM((_G, _DIM), jnp.float32),
        pltpu.SemaphoreType.DMA,
    ],
)
def _sc_embedding_lookup(idx_hbm, table_hbm, out_hbm, idx_v, rows_v, sem):
    wid = lax.axis_index("s") * _NC + lax.axis_index("c")
    base = wid * _BPW

    pltpu.sync_copy(idx_hbm.at[wid], idx_v)

    def clamp_row(j, _):
        def clamp_vec(i, _):
            sl = pl.ds(i * _LANES, _LANES)
            v = idx_v[j, sl]
            idx_v[j, sl] = jnp.minimum(
                jnp.maximum(v, 0), _NUM_EMBEDDINGS - 1
            )
            return 0
        return lax.fori_loop(0, _G // _LANES, clamp_vec, 0)

    lax.fori_loop(0, _NG, clamp_row, 0)

    def group(g, _):
        pltpu.async_copy(table_hbm.at[idx_v.at[g]], rows_v, sem).wait()
        pltpu.sync_copy(rows_v, out_hbm.at[pl.ds(base + g * _G, _G)])
        return 0

    lax.fori_loop(0, _NG, group, 0)


def kernel(idx, table):
    flat = idx.reshape(_NW, _NG, _G).astype(jnp.int32)
    out = _sc_embedding_lookup(flat, table)
    return out.reshape(_B, _S, _DIM)


# speedup vs baseline: 1.6811x; 1.6811x over previous
import functools

import jax
import jax.numpy as jnp
from jax import lax
from jax.experimental import pallas as pl
from jax.experimental.pallas import tpu as pltpu
from jax.experimental.pallas import tpu_sc as plsc

_NUM_EMBEDDINGS = 1000000
_DIM = 64
_B, _S = 16384, 50
_TOTAL = _B * _S
_NC, _NS = 2, 16
_NW = _NC * _NS
_BPW = _TOTAL // _NW
_G = 128
_NG = _BPW // _G
_LANES = 16

_mesh = plsc.VectorSubcoreMesh(core_axis_name="c", subcore_axis_name="s")


@functools.partial(
    pl.kernel,
    mesh=_mesh,
    out_type=jax.ShapeDtypeStruct((_TOTAL, _DIM), jnp.float32),
    scratch_types=[
        pltpu.VMEM((_NG, _G), jnp.int32),
        pltpu.VMEM((_G, _DIM), jnp.float32),
        pltpu.SemaphoreType.DMA,
    ],
    compiler_params=pltpu.CompilerParams(use_tc_tiling_on_sc=False),
)
def _sc_embedding_lookup(idx_hbm, table_hbm, out_hbm, idx_v, rows_v, sem):
    wid = lax.axis_index("s") * _NC + lax.axis_index("c")
    base = wid * _BPW

    pltpu.sync_copy(idx_hbm.at[wid], idx_v)

    def clamp_row(j, _):
        def clamp_vec(i, _):
            sl = pl.ds(i * _LANES, _LANES)
            v = idx_v[j, sl]
            idx_v[j, sl] = jnp.minimum(
                jnp.maximum(v, 0), _NUM_EMBEDDINGS - 1
            )
            return 0
        return lax.fori_loop(0, _G // _LANES, clamp_vec, 0)

    lax.fori_loop(0, _NG, clamp_row, 0)

    def group(g, _):
        pltpu.async_copy(table_hbm.at[idx_v.at[g]], rows_v, sem).wait()
        pltpu.sync_copy(rows_v, out_hbm.at[pl.ds(base + g * _G, _G)])
        return 0

    lax.fori_loop(0, _NG, group, 0)


def kernel(idx, table):
    flat = idx.reshape(_NW, _NG, _G).astype(jnp.int32)
    out = _sc_embedding_lookup(flat, table)
    return out.reshape(_B, _S, _DIM)


# speedup vs baseline: 1.8734x; 1.1144x over previous
import functools

import jax
import jax.numpy as jnp
from jax import lax
from jax.experimental import pallas as pl
from jax.experimental.pallas import tpu as pltpu
from jax.experimental.pallas import tpu_sc as plsc

_NUM_EMBEDDINGS = 1000000
_DIM = 64
_B, _S = 16384, 50
_TOTAL = _B * _S
_NC, _NS = 2, 16
_NW = _NC * _NS
_BPW = _TOTAL // _NW
_G = 128
_NG = _BPW // _G
_KB = 5
_NSUP = _NG // _KB
_NPAIR = _NSUP // 2
_LANES = 16

_mesh = plsc.VectorSubcoreMesh(core_axis_name="c", subcore_axis_name="s")


@functools.partial(
    pl.kernel,
    mesh=_mesh,
    out_type=jax.ShapeDtypeStruct((_TOTAL // _G, _G, _DIM), jnp.float32),
    scratch_types=[
        pltpu.VMEM((_NG, _G), jnp.int32),
        pltpu.VMEM((2, _KB, _G, _DIM), jnp.float32),
        pltpu.SemaphoreType.DMA,
        pltpu.SemaphoreType.DMA,
        pltpu.SemaphoreType.DMA,
        pltpu.SemaphoreType.DMA,
    ],
    compiler_params=pltpu.CompilerParams(use_tc_tiling_on_sc=False),
)
def _sc_embedding_lookup(idx_hbm, table_hbm, out_hbm, idx_v, rows_v,
                         gsem0, gsem1, ssem0, ssem1):
    wid = lax.axis_index("s") * _NC + lax.axis_index("c")
    out_base = wid * _NG

    pltpu.sync_copy(idx_hbm.at[wid], idx_v)

    def clamp_chunk(c):
        for j in range(_KB):
            for i in range(_G // _LANES):
                sl = pl.ds(i * _LANES, _LANES)
                v = idx_v[c * _KB + j, sl]
                idx_v[c * _KB + j, sl] = jnp.minimum(
                    jnp.maximum(v, 0), _NUM_EMBEDDINGS - 1
                )

    def fire_gathers(c, h, gsem):
        clamp_chunk(c)
        for j in range(_KB):
            pltpu.async_copy(
                table_hbm.at[idx_v.at[c * _KB + j]], rows_v.at[h, j], gsem
            )

    def drain_gathers(c, h, gsem):
        for j in range(_KB):
            pltpu.make_async_copy(
                table_hbm.at[idx_v.at[c * _KB + j]], rows_v.at[h, j], gsem
            ).wait()

    def store_copy(c, h, ssem):
        return pltpu.make_async_copy(
            rows_v.at[h], out_hbm.at[pl.ds(out_base + c * _KB, _KB)], ssem
        )

    fire_gathers(0, 0, gsem0)

    def pair(t, _):
        c0 = 2 * t
        c1 = 2 * t + 1

        @pl.when(t > 0)
        def _():
            store_copy(c1 - 2, 1, ssem1).wait()

        fire_gathers(c1, 1, gsem1)
        drain_gathers(c0, 0, gsem0)
        store_copy(c0, 0, ssem0).start()

        @pl.when(t + 1 < _NPAIR)
        def _():
            store_copy(c0, 0, ssem0).wait()
            fire_gathers(c0 + 2, 0, gsem0)

        drain_gathers(c1, 1, gsem1)
        store_copy(c1, 1, ssem1).start()
        return 0

    lax.fori_loop(0, _NPAIR, pair, 0)

    store_copy(_NSUP - 2, 0, ssem0).wait()
    store_copy(_NSUP - 1, 1, ssem1).wait()


def kernel(idx, table):
    flat = idx.reshape(_NW, _NG, _G).astype(jnp.int32)
    out = _sc_embedding_lookup(flat, table)
    return out.reshape(_B, _S, _DIM)
